# Initial kernel scaffold; baseline (speedup 1.0000x reference)
import jax
import jax.numpy as jnp
from jax.experimental import pallas as pl


def kernel(pc1, pc2):
    raise NotImplementedError("write your pallas kernel here")


# speedup vs baseline: 1.8203x; 1.8203x over previous
import jax
import jax.numpy as jnp
from jax.experimental import pallas as pl
from jax.experimental.pallas import tpu as pltpu

_TI = 512


def _chamfer_kernel(pc1_ref, pc2t_ref, out_ref, colmin_ref):
    b = pl.program_id(0)
    i = pl.program_id(1)
    ni = pl.num_programs(1)

    a = pc1_ref[0]
    bt = pc2t_ref[0]

    d2 = (a[:, 0:1] - bt[0:1, :]) ** 2
    d2 += (a[:, 1:2] - bt[1:2, :]) ** 2
    d2 += (a[:, 2:3] - bt[2:3, :]) ** 2

    row_min = jnp.min(d2, axis=1, keepdims=True)
    col_partial = jnp.min(d2, axis=0, keepdims=True)

    @pl.when(jnp.logical_and(b == 0, i == 0))
    def _():
        out_ref[...] = jnp.zeros_like(out_ref)

    @pl.when(i == 0)
    def _():
        colmin_ref[...] = col_partial

    @pl.when(i > 0)
    def _():
        colmin_ref[...] = jnp.minimum(colmin_ref[...], col_partial)

    n = pc2t_ref.shape[2]
    batches = pl.num_programs(0)
    scale = 1.0 / (2.0 * n * batches)

    acc = (jnp.sum(jnp.sqrt(row_min)) * scale).reshape(1, 1)
    out_ref[...] += acc

    @pl.when(i == ni - 1)
    def _():
        out_ref[...] += (jnp.sum(jnp.sqrt(colmin_ref[...])) * scale).reshape(1, 1)


def kernel(pc1, pc2):
    batches, n, d = pc1.shape
    pc2t = jnp.swapaxes(pc2, 1, 2)
    ni = n // _TI
    out = pl.pallas_call(
        _chamfer_kernel,
        grid=(batches, ni),
        in_specs=[
            pl.BlockSpec((1, _TI, d), lambda b, i: (b, i, 0)),
            pl.BlockSpec((1, d, n), lambda b, i: (b, 0, 0)),
        ],
        out_specs=pl.BlockSpec((1, 1), lambda b, i: (0, 0)),
        out_shape=jax.ShapeDtypeStruct((1, 1), jnp.float32),
        scratch_shapes=[pltpu.VMEM((1, n), jnp.float32)],
    )(pc1, pc2t)
    return out[0, 0]


# speedup vs baseline: 3.0801x; 1.6920x over previous
import jax
import jax.numpy as jnp
from jax.experimental import pallas as pl
from jax.experimental.pallas import tpu as pltpu

_TI = 4096


def _chamfer_kernel(pc1_ref, pc2t_ref, out_ref, colmin_ref):
    i = pl.program_id(1)
    ni = pl.num_programs(1)
    n = pc2t_ref.shape[2]

    a = pc1_ref[0]
    bt = pc2t_ref[0]

    asq = jnp.sum(a * a, axis=1, keepdims=True)
    bsq = jnp.sum(bt * bt, axis=0, keepdims=True)

    a2 = -2.0 * a
    a2h = a2.astype(jnp.bfloat16)
    a2l = (a2 - a2h.astype(jnp.float32)).astype(jnp.bfloat16)
    bth = bt.astype(jnp.bfloat16)
    btl = (bt - bth.astype(jnp.float32)).astype(jnp.bfloat16)
    sh = asq.astype(jnp.bfloat16)
    sl = (asq - sh.astype(jnp.float32)).astype(jnp.bfloat16)
    th = bsq.astype(jnp.bfloat16)
    tl = (bsq - th.astype(jnp.float32)).astype(jnp.bfloat16)
    ones_a = jnp.ones_like(sh)
    ones_b = jnp.ones_like(th)
    a_aug = jnp.concatenate(
        [a2h, a2l, a2h, a2l, sh, sl, ones_a, ones_a], axis=1)
    b_aug = jnp.concatenate(
        [bth, bth, btl, btl, ones_b, ones_b, th, tl], axis=0)

    d2 = jnp.dot(a_aug, b_aug, preferred_element_type=jnp.float32)

    row_min = jnp.min(d2, axis=1, keepdims=True)
    col_partial = jnp.min(d2, axis=0, keepdims=True)

    @pl.when(i == 0)
    def _():
        out_ref[...] = jnp.zeros_like(out_ref)
        colmin_ref[...] = col_partial

    @pl.when(i > 0)
    def _():
        colmin_ref[...] = jnp.minimum(colmin_ref[...], col_partial)

    scale = 1.0 / (2.0 * n)
    row_acc = jnp.sum(jnp.sqrt(jnp.maximum(row_min, 0.0))) * scale
    out_ref[...] += row_acc.reshape(1, 1, 1)

    @pl.when(i == ni - 1)
    def _():
        col_acc = jnp.sum(jnp.sqrt(jnp.maximum(colmin_ref[...], 0.0))) * scale
        out_ref[...] += col_acc.reshape(1, 1, 1)


def kernel(pc1, pc2):
    batches, n, d = pc1.shape
    pc2t = jnp.swapaxes(pc2, 1, 2)
    ni = n // _TI
    out = pl.pallas_call(
        _chamfer_kernel,
        grid=(batches, ni),
        in_specs=[
            pl.BlockSpec((1, _TI, d), lambda b, i: (b, i, 0)),
            pl.BlockSpec((1, d, n), lambda b, i: (b, 0, 0)),
        ],
        out_specs=pl.BlockSpec((1, 1, 1), lambda b, i: (b, 0, 0)),
        out_shape=jax.ShapeDtypeStruct((batches, 1, 1), jnp.float32),
        scratch_shapes=[pltpu.VMEM((1, n), jnp.float32)],
        compiler_params=pltpu.CompilerParams(
            dimension_semantics=("parallel", "arbitrary"),
        ),
    )(pc1, pc2t)
    return jnp.mean(out)


# speedup vs baseline: 5.1465x; 1.6709x over previous
import jax
import jax.numpy as jnp
from jax import lax
from jax.experimental import pallas as pl
from jax.experimental.pallas import tpu as pltpu


def _chamfer_kernel(pc1t_ref, pc2t_ref, out_ref):
    n = pc2t_ref.shape[2]

    at = pc1t_ref[0]
    bt = pc2t_ref[0]

    asq = jnp.sum(at * at, axis=0, keepdims=True)
    bsq = jnp.sum(bt * bt, axis=0, keepdims=True)

    a2 = -2.0 * at
    ah = a2.astype(jnp.bfloat16)
    al = (a2 - ah.astype(jnp.float32)).astype(jnp.bfloat16)
    sh = asq.astype(jnp.bfloat16)
    sl = (asq - sh.astype(jnp.float32)).astype(jnp.bfloat16)
    ones_a = jnp.ones_like(sh)
    bh = bt.astype(jnp.bfloat16)
    bl = (bt - bh.astype(jnp.float32)).astype(jnp.bfloat16)
    th = bsq.astype(jnp.bfloat16)
    tl = (bsq - th.astype(jnp.float32)).astype(jnp.bfloat16)
    ones_b = jnp.ones_like(th)

    a_augt = jnp.concatenate(
        [ah, al, ah, al, sh, sl, ones_a, ones_a], axis=0)
    b_aug = jnp.concatenate(
        [bh, bh, bl, bl, ones_b, ones_b, th, tl], axis=0)

    d2 = lax.dot_general(a_augt, b_aug, (((0,), (0,)), ((), ())),
                         preferred_element_type=jnp.float32)

    row_min = jnp.min(d2, axis=1, keepdims=True)
    col_min = jnp.min(d2, axis=0, keepdims=True)

    scale = 1.0 / (2.0 * n)
    acc = (jnp.sum(jnp.sqrt(jnp.maximum(row_min, 0.0)))
           + jnp.sum(jnp.sqrt(jnp.maximum(col_min, 0.0)))) * scale
    out_ref[...] = acc.reshape(1, 1, 1)


def kernel(pc1, pc2):
    batches, n, d = pc1.shape
    pc1t = jnp.swapaxes(pc1, 1, 2)
    pc2t = jnp.swapaxes(pc2, 1, 2)
    out = pl.pallas_call(
        _chamfer_kernel,
        grid=(batches,),
        in_specs=[
            pl.BlockSpec((1, d, n), lambda b: (b, 0, 0)),
            pl.BlockSpec((1, d, n), lambda b: (b, 0, 0)),
        ],
        out_specs=pl.BlockSpec((1, 1, 1), lambda b: (b, 0, 0)),
        out_shape=jax.ShapeDtypeStruct((batches, 1, 1), jnp.float32),
        compiler_params=pltpu.CompilerParams(
            dimension_semantics=("parallel",),
        ),
    )(pc1t, pc2t)
    return jnp.mean(out)
